# Initial kernel scaffold; baseline (speedup 1.0000x reference)
import jax
import jax.numpy as jnp
from jax.experimental import pallas as pl


def kernel(nh, W_nf, W_attn, W_out, b_out, edge_y, edge_index):
    raise NotImplementedError("write your pallas kernel here")


# speedup vs baseline: 39.0738x; 39.0738x over previous
import functools

import jax
import jax.numpy as jnp
from jax import lax
from jax.experimental import pallas as pl
from jax.experimental.pallas import tpu as pltpu
from jax.experimental.pallas import tpu_sc as plsc

N = 10000
F = 128
E = 320000
T = 5
NSEG = 50176
NC = 2
NS = 16
NW = NC * NS
EPW = E // NW
K = 80
NCHUNK = EPW // K
SEG_SL = NSEG // NS
ROW_SL = N // NS

_mesh = plsc.VectorSubcoreMesh(
    core_axis_name="c", subcore_axis_name="s", num_cores=NC, num_subcores=NS)


def _tc1_body(nh_ref, wnf_ref, a_ref, g_ref, pq_ref):
  g = lax.dot_general(nh_ref[...], wnf_ref[...],
                      (((1,), (1,)), ((), ())),
                      preferred_element_type=jnp.float32)
  g_ref[...] = g
  pq_ref[...] = lax.dot_general(g, a_ref[...],
                                (((1,), (0,)), ((), ())),
                                preferred_element_type=jnp.float32)


def _tc1(nh, wnf, a):
  blk = 1000
  return pl.pallas_call(
      _tc1_body,
      grid=(N // blk,),
      in_specs=[
          pl.BlockSpec((blk, F), lambda i: (i, 0)),
          pl.BlockSpec((F, F), lambda i: (0, 0)),
          pl.BlockSpec((F, 2), lambda i: (0, 0)),
      ],
      out_specs=[
          pl.BlockSpec((blk, F), lambda i: (i, 0)),
          pl.BlockSpec((blk, 2), lambda i: (i, 0)),
      ],
      out_shape=[
          jax.ShapeDtypeStruct((N, F), jnp.float32),
          jax.ShapeDtypeStruct((N, 2), jnp.float32),
      ],
  )(nh, wnf, a)


def _sca_body(src_hbm, dst_hbm, ty_hbm, pq_hbm,
              w_hbm, s_out,
              pq_v, src_v, dst_v, ty_v, w_v, key_v, zb_v, s_sh):
  cid = lax.axis_index("c")
  sid = lax.axis_index("s")
  wid = sid * NC + cid

  def _z(i, _):
    zb_v[pl.ds(i * 16, 16)] = jnp.zeros((16,), jnp.float32)
    return _
  lax.fori_loop(0, SEG_SL // 16, _z, None)
  pltpu.sync_copy(zb_v, s_sh.at[pl.ds(sid * SEG_SL, SEG_SL)])

  pltpu.sync_copy(pq_hbm, pq_v)
  plsc.subcore_barrier()

  def _chunk(c, _):
    base = wid * EPW + c * K
    pltpu.sync_copy(src_hbm.at[pl.ds(base, K)], src_v)
    pltpu.sync_copy(dst_hbm.at[pl.ds(base, K)], dst_v)
    pltpu.sync_copy(ty_hbm.at[pl.ds(base, K)], ty_v)
    for j in range(K // 16):
      sl = pl.ds(j * 16, 16)
      s16 = src_v[sl]
      d16 = dst_v[sl]
      t16 = ty_v[sl]
      pe = plsc.load_gather(pq_v, [s16 * 2])
      qe = plsc.load_gather(pq_v, [d16 * 2 + 1])
      ef = pe + qe
      ef = jnp.where(ef > 0.0, ef, 0.2 * ef)
      w_v[sl] = jnp.exp(ef)
      key_v[sl] = d16 * T + t16
    pltpu.sync_copy(w_v, w_hbm.at[pl.ds(base, K)])
    pltpu.sync_copy(w_v, s_sh.at[key_v], add=True)
    return _
  lax.fori_loop(0, NCHUNK, _chunk, None)

  plsc.subcore_barrier()
  pltpu.sync_copy(s_sh.at[pl.ds(sid * SEG_SL, SEG_SL)], zb_v)
  pltpu.sync_copy(zb_v, s_out.at[pl.ds(cid * NSEG + sid * SEG_SL, SEG_SL)])


_sca = pl.kernel(
    _sca_body,
    out_type=(
        jax.ShapeDtypeStruct((E,), jnp.float32),
        jax.ShapeDtypeStruct((NC * NSEG,), jnp.float32),
    ),
    mesh=_mesh,
    compiler_params=pltpu.CompilerParams(needs_layout_passes=False),
    scratch_types=[
        pltpu.VMEM((N * 2,), jnp.float32),
        pltpu.VMEM((K,), jnp.int32),
        pltpu.VMEM((K,), jnp.int32),
        pltpu.VMEM((K,), jnp.int32),
        pltpu.VMEM((K,), jnp.float32),
        pltpu.VMEM((K,), jnp.int32),
        pltpu.VMEM((SEG_SL,), jnp.float32),
        pltpu.VMEM_SHARED((NSEG,), jnp.float32),
    ],
)


SEG_W = NSEG // NW


def _scs_body(sp_hbm, s_out, b0_v, b1_v):
  cid = lax.axis_index("c")
  sid = lax.axis_index("s")
  wid = sid * NC + cid
  base = wid * SEG_W
  pltpu.sync_copy(sp_hbm.at[pl.ds(base, SEG_W)], b0_v)
  pltpu.sync_copy(sp_hbm.at[pl.ds(NSEG + base, SEG_W)], b1_v)
  def _acc(i, _):
    sl = pl.ds(i * 16, 16)
    b0_v[sl] = b0_v[sl] + b1_v[sl]
    return _
  lax.fori_loop(0, SEG_W // 16, _acc, None)
  pltpu.sync_copy(b0_v, s_out.at[pl.ds(base, SEG_W)])


_scs = pl.kernel(
    _scs_body,
    out_type=jax.ShapeDtypeStruct((NSEG,), jnp.float32),
    mesh=_mesh,
    compiler_params=pltpu.CompilerParams(needs_layout_passes=False),
    scratch_types=[
        pltpu.VMEM((SEG_W,), jnp.float32),
        pltpu.VMEM((SEG_W,), jnp.float32),
    ],
)


def _scb_body(src_hbm, dst_hbm, ty_hbm, w_hbm, s_hbm, g_hbm,
              h_out,
              rows_v, src_v, dst_v, ty_v, w_v, key_v, sv_v, al_v,
              sem, sem2, h_sh):
  cid = lax.axis_index("c")
  sid = lax.axis_index("s")
  wid = sid * NC + cid

  def _z(i, _):
    r = i // (F // 16)
    cc = i % (F // 16)
    rows_v[r, pl.ds(cc * 16, 16)] = jnp.zeros((16,), jnp.float32)
    return _
  lax.fori_loop(0, K * (F // 16), _z, None)
  NB = N // K
  def _zh(i, _):
    b = sid + i * NS
    @pl.when(b < NB)
    def _():
      pltpu.sync_copy(rows_v, h_sh.at[pl.ds(b * K, K)])
    return _
  lax.fori_loop(0, (NB + NS - 1) // NS, _zh, None)
  plsc.subcore_barrier()

  def _chunk(c, _):
    base = wid * EPW + c * K
    pltpu.sync_copy(src_hbm.at[pl.ds(base, K)], src_v)
    pltpu.sync_copy(dst_hbm.at[pl.ds(base, K)], dst_v)
    pltpu.sync_copy(ty_hbm.at[pl.ds(base, K)], ty_v)
    pltpu.sync_copy(w_hbm.at[pl.ds(base, K)], w_v)
    d1 = pltpu.async_copy(g_hbm.at[src_v], rows_v, sem)
    for j in range(K // 16):
      sl = pl.ds(j * 16, 16)
      key_v[sl] = dst_v[sl] * T + ty_v[sl]
    d2 = pltpu.async_copy(s_hbm.at[key_v], sv_v, sem2)
    d1.wait()
    d2.wait()
    for j in range(K // 16):
      sl = pl.ds(j * 16, 16)
      al_v[sl] = w_v[sl] / sv_v[sl]
    def _scale(j, _2):
      al16 = al_v[pl.ds(j * 16, 16)]
      for l in range(16):
        a = al16[l]
        e = j * 16 + l
        for cc in range(F // 16):
          csl = pl.ds(cc * 16, 16)
          rows_v[e, csl] = rows_v[e, csl] * a
      return _2
    lax.fori_loop(0, K // 16, _scale, None)
    pltpu.sync_copy(rows_v, h_sh.at[dst_v], add=True)
    return _
  lax.fori_loop(0, NCHUNK, _chunk, None)

  plsc.subcore_barrier()
  def _pub(i, _):
    b = sid + i * NS
    @pl.when(b < NB)
    def _():
      pltpu.sync_copy(h_sh.at[pl.ds(b * K, K)], rows_v)
      pltpu.sync_copy(rows_v, h_out.at[cid, pl.ds(b * K, K)])
    return _
  lax.fori_loop(0, (NB + NS - 1) // NS, _pub, None)


_scb = pl.kernel(
    _scb_body,
    out_type=jax.ShapeDtypeStruct((NC, N, F), jnp.float32),
    mesh=_mesh,
    compiler_params=pltpu.CompilerParams(needs_layout_passes=False),
    scratch_types=[
        pltpu.VMEM((K, F), jnp.float32),
        pltpu.VMEM((K,), jnp.int32),
        pltpu.VMEM((K,), jnp.int32),
        pltpu.VMEM((K,), jnp.int32),
        pltpu.VMEM((K,), jnp.float32),
        pltpu.VMEM((K,), jnp.int32),
        pltpu.VMEM((K,), jnp.float32),
        pltpu.VMEM((K,), jnp.float32),
        pltpu.SemaphoreType.DMA,
        pltpu.SemaphoreType.DMA,
        pltpu.VMEM_SHARED((N, F), jnp.float32),
    ],
)


def _tc2_body(nh_ref, hp_ref, wo_ref, b_ref, out_ref):
  wo = wo_ref[...]
  h = hp_ref[0] + hp_ref[1]
  acc = lax.dot_general(nh_ref[...], wo[:, :F],
                        (((1,), (1,)), ((), ())),
                        preferred_element_type=jnp.float32)
  acc = acc + lax.dot_general(h, wo[:, F:],
                              (((1,), (1,)), ((), ())),
                              preferred_element_type=jnp.float32)
  out_ref[...] = jnp.maximum(acc + b_ref[...], 0.0)


def _tc2(nh, h_parts, wo, b2d):
  blk = 1000
  return pl.pallas_call(
      _tc2_body,
      grid=(N // blk,),
      in_specs=[
          pl.BlockSpec((blk, F), lambda i: (i, 0)),
          pl.BlockSpec((NC, blk, F), lambda i: (0, i, 0)),
          pl.BlockSpec((F, 2 * F), lambda i: (0, 0)),
          pl.BlockSpec((1, F), lambda i: (0, 0)),
      ],
      out_specs=pl.BlockSpec((blk, F), lambda i: (i, 0)),
      out_shape=jax.ShapeDtypeStruct((N, F), jnp.float32),
  )(nh, h_parts, wo, b2d)


@jax.jit
def kernel(nh, W_nf, W_attn, W_out, b_out, edge_y, edge_index):
  src = edge_index[0].astype(jnp.int32)
  dst = edge_index[1].astype(jnp.int32)
  ty = edge_y.astype(jnp.int32)
  a = jnp.stack([W_attn[0, :F], W_attn[0, F:]], axis=1)
  g, pq = _tc1(nh, W_nf, a)
  w, s_parts = _sca(src, dst, ty, pq.reshape(-1))
  s_total = _scs(s_parts)
  h_parts = _scb(src, dst, ty, w, s_total, g)
  return _tc2(nh, h_parts, W_out, b_out.reshape(1, F))


# speedup vs baseline: 109.5839x; 2.8045x over previous
import functools

import jax
import jax.numpy as jnp
from jax import lax
from jax.experimental import pallas as pl
from jax.experimental.pallas import tpu as pltpu
from jax.experimental.pallas import tpu_sc as plsc

N = 10000
F = 128
E = 320000
T = 5
NSEG = 50176
NC = 2
NS = 16
NW = NC * NS
EPW = E // NW
K = 80
NCHUNK = EPW // K
SEG_SL = NSEG // NS
ROW_SL = N // NS

_mesh = plsc.VectorSubcoreMesh(
    core_axis_name="c", subcore_axis_name="s", num_cores=NC, num_subcores=NS)


def _tc1_body(nh_ref, wnf_ref, a_ref, g_ref, pq_ref):
  g = lax.dot_general(nh_ref[...], wnf_ref[...],
                      (((1,), (1,)), ((), ())),
                      preferred_element_type=jnp.float32)
  g_ref[...] = g
  pq_ref[...] = lax.dot_general(g, a_ref[...],
                                (((1,), (0,)), ((), ())),
                                preferred_element_type=jnp.float32)


def _tc1(nh, wnf, a):
  blk = 1000
  return pl.pallas_call(
      _tc1_body,
      grid=(N // blk,),
      in_specs=[
          pl.BlockSpec((blk, F), lambda i: (i, 0)),
          pl.BlockSpec((F, F), lambda i: (0, 0)),
          pl.BlockSpec((F, 2), lambda i: (0, 0)),
      ],
      out_specs=[
          pl.BlockSpec((blk, F), lambda i: (i, 0)),
          pl.BlockSpec((blk, 2), lambda i: (i, 0)),
      ],
      out_shape=[
          jax.ShapeDtypeStruct((N, F), jnp.float32),
          jax.ShapeDtypeStruct((N, 2), jnp.float32),
      ],
  )(nh, wnf, a)


KA = 400
NCH_A = EPW // KA
KSUB = 80


def _sca_body(src_hbm, dst_hbm, ty_hbm, pq_hbm,
              w_hbm, s_out,
              pq_v, src_v, dst_v, ty_v, w_v, zb_v,
              kb0, kb1, kb2, kb3, kb4, sem_l, sem_s, s_sh):
  cid = lax.axis_index("c")
  sid = lax.axis_index("s")
  wid = sid * NC + cid
  kbufs = [kb0, kb1, kb2, kb3, kb4]

  def _z(i, _):
    zb_v[pl.ds(i * 16, 16)] = jnp.zeros((16,), jnp.float32)
    return _
  lax.fori_loop(0, SEG_SL // 16, _z, None)
  pltpu.sync_copy(zb_v, s_sh.at[pl.ds(sid * SEG_SL, SEG_SL)])

  pltpu.sync_copy(pq_hbm, pq_v)
  plsc.subcore_barrier()

  def _chunk(c, _):
    base = wid * EPW + c * KA
    d1 = pltpu.async_copy(src_hbm.at[pl.ds(base, KA)], src_v, sem_l)
    d2 = pltpu.async_copy(dst_hbm.at[pl.ds(base, KA)], dst_v, sem_l)
    d3 = pltpu.async_copy(ty_hbm.at[pl.ds(base, KA)], ty_v, sem_l)
    d1.wait()
    d2.wait()
    d3.wait()
    for j in range(KA // 16):
      sl = pl.ds(j * 16, 16)
      s16 = src_v[sl]
      d16 = dst_v[sl]
      t16 = ty_v[sl]
      pe = plsc.load_gather(pq_v, [s16 * 2])
      qe = plsc.load_gather(pq_v, [d16 * 2 + 1])
      ef = pe + qe
      ef = jnp.where(ef > 0.0, ef, 0.2 * ef)
      w_v[sl] = jnp.exp(ef)
      kbufs[j // (KSUB // 16)][pl.ds((j % (KSUB // 16)) * 16, 16)] = (
          d16 * T + t16)
    dw = pltpu.async_copy(w_v, w_hbm.at[pl.ds(base, KA)], sem_l)
    dss = [
        pltpu.async_copy(w_v.at[pl.ds(i * KSUB, KSUB)],
                         s_sh.at[kbufs[i]], sem_s, add=True)
        for i in range(KA // KSUB)
    ]
    dw.wait()
    for d in dss:
      d.wait()
    return _
  lax.fori_loop(0, NCH_A, _chunk, None)

  plsc.subcore_barrier()
  pltpu.sync_copy(s_sh.at[pl.ds(sid * SEG_SL, SEG_SL)], zb_v)
  pltpu.sync_copy(zb_v, s_out.at[pl.ds(cid * NSEG + sid * SEG_SL, SEG_SL)])


_sca = pl.kernel(
    _sca_body,
    out_type=(
        jax.ShapeDtypeStruct((E,), jnp.float32),
        jax.ShapeDtypeStruct((NC * NSEG,), jnp.float32),
    ),
    mesh=_mesh,
    compiler_params=pltpu.CompilerParams(needs_layout_passes=False),
    scratch_types=[
        pltpu.VMEM((N * 2,), jnp.float32),
        pltpu.VMEM((KA,), jnp.int32),
        pltpu.VMEM((KA,), jnp.int32),
        pltpu.VMEM((KA,), jnp.int32),
        pltpu.VMEM((KA,), jnp.float32),
        pltpu.VMEM((SEG_SL,), jnp.float32),
        pltpu.VMEM((KSUB,), jnp.int32),
        pltpu.VMEM((KSUB,), jnp.int32),
        pltpu.VMEM((KSUB,), jnp.int32),
        pltpu.VMEM((KSUB,), jnp.int32),
        pltpu.VMEM((KSUB,), jnp.int32),
        pltpu.SemaphoreType.DMA,
        pltpu.SemaphoreType.DMA,
        pltpu.VMEM_SHARED((NSEG,), jnp.float32),
    ],
)


SEG_W = NSEG // NW


def _scs_body(sp_hbm, s_out, b0_v, b1_v):
  cid = lax.axis_index("c")
  sid = lax.axis_index("s")
  wid = sid * NC + cid
  base = wid * SEG_W
  pltpu.sync_copy(sp_hbm.at[pl.ds(base, SEG_W)], b0_v)
  pltpu.sync_copy(sp_hbm.at[pl.ds(NSEG + base, SEG_W)], b1_v)
  def _acc(i, _):
    sl = pl.ds(i * 16, 16)
    b0_v[sl] = b0_v[sl] + b1_v[sl]
    return _
  lax.fori_loop(0, SEG_W // 16, _acc, None)
  pltpu.sync_copy(b0_v, s_out.at[pl.ds(base, SEG_W)])


_scs = pl.kernel(
    _scs_body,
    out_type=jax.ShapeDtypeStruct((NSEG,), jnp.float32),
    mesh=_mesh,
    compiler_params=pltpu.CompilerParams(needs_layout_passes=False),
    scratch_types=[
        pltpu.VMEM((SEG_W,), jnp.float32),
        pltpu.VMEM((SEG_W,), jnp.float32),
    ],
)


def _scb_body(src_hbm, dst_hbm, ty_hbm, w_hbm, s_hbm, g_hbm,
              h_out,
              rows0, rows1, src0, src1, dst0, dst1, ty0, ty1,
              w0, w1, key0, key1, sv0, sv1, al0, al1,
              seml0, seml1, semg0, semg1, sems0, sems1, h_sh):
  cid = lax.axis_index("c")
  sid = lax.axis_index("s")
  wid = sid * NC + cid
  rows = [rows0, rows1]
  srcs = [src0, src1]
  dsts = [dst0, dst1]
  tys = [ty0, ty1]
  ws = [w0, w1]
  keys = [key0, key1]
  svs = [sv0, sv1]
  als = [al0, al1]
  seml = [seml0, seml1]
  semg = [semg0, semg1]
  sems = [sems0, sems1]

  def _z(i, _):
    r = i // (F // 16)
    cc = i % (F // 16)
    rows0[r, pl.ds(cc * 16, 16)] = jnp.zeros((16,), jnp.float32)
    return _
  lax.fori_loop(0, K * (F // 16), _z, None)
  NB = N // K
  def _zh(i, _):
    b = sid + i * NS
    @pl.when(b < NB)
    def _():
      pltpu.sync_copy(rows0, h_sh.at[pl.ds(b * K, K)])
    return _
  lax.fori_loop(0, (NB + NS - 1) // NS, _zh, None)
  plsc.subcore_barrier()

  def _issue_loads(c, b):
    base = wid * EPW + c * K
    pltpu.async_copy(src_hbm.at[pl.ds(base, K)], srcs[b], seml[b])
    pltpu.async_copy(dst_hbm.at[pl.ds(base, K)], dsts[b], seml[b])
    pltpu.async_copy(ty_hbm.at[pl.ds(base, K)], tys[b], seml[b])
    pltpu.async_copy(w_hbm.at[pl.ds(base, K)], ws[b], seml[b])

  def _drain_loads(b):
    pltpu.make_async_copy(src_hbm.at[pl.ds(0, K)], srcs[b], seml[b]).wait()
    pltpu.make_async_copy(dst_hbm.at[pl.ds(0, K)], dsts[b], seml[b]).wait()
    pltpu.make_async_copy(ty_hbm.at[pl.ds(0, K)], tys[b], seml[b]).wait()
    pltpu.make_async_copy(w_hbm.at[pl.ds(0, K)], ws[b], seml[b]).wait()

  def _issue_gathers(b):
    for j in range(K // 16):
      sl = pl.ds(j * 16, 16)
      keys[b][sl] = dsts[b][sl] * T + tys[b][sl]
    pltpu.async_copy(g_hbm.at[srcs[b]], rows[b], semg[b])
    pltpu.async_copy(s_hbm.at[keys[b]], svs[b], semg[b])

  def _drain_gathers(b):
    pltpu.make_async_copy(g_hbm.at[pl.ds(0, K)], rows[b], semg[b]).wait()
    pltpu.make_async_copy(s_hbm.at[pl.ds(0, K)], svs[b], semg[b]).wait()

  def _drain_scatter(b):
    pltpu.make_async_copy(g_hbm.at[pl.ds(0, K)], rows[b], sems[b]).wait()

  def _compute_scatter(b):
    for j in range(K // 16):
      sl = pl.ds(j * 16, 16)
      als[b][sl] = ws[b][sl] / svs[b][sl]
    def _scale(j, _2):
      al16 = als[b][pl.ds(j * 16, 16)]
      for l in range(16):
        a = al16[l]
        e = j * 16 + l
        for cc in range(F // 16):
          csl = pl.ds(cc * 16, 16)
          rows[b][e, csl] = rows[b][e, csl] * a
      return _2
    lax.fori_loop(0, K // 16, _scale, None)
    pltpu.async_copy(rows[b], h_sh.at[dsts[b]], sems[b], add=True)

  for b in range(2):
    _issue_loads(b, b)
    _drain_loads(b)
    _issue_gathers(b)

  def _main(i, _):
    for b in range(2):
      c = 2 * i + b
      @pl.when(c < NCHUNK)
      def _():
        _drain_gathers(b)
        _compute_scatter(b)
        cn = c + 2
        @pl.when(cn < NCHUNK)
        def _():
          _issue_loads(cn, b)
          _drain_loads(b)
          _drain_scatter(b)
          _issue_gathers(b)
    return _
  lax.fori_loop(0, (NCHUNK + 1) // 2, _main, None)

  _drain_scatter((NCHUNK - 1) % 2)
  _drain_scatter(NCHUNK % 2)

  plsc.subcore_barrier()
  def _pub(i, _):
    b = sid + i * NS
    @pl.when(b < NB)
    def _():
      pltpu.sync_copy(h_sh.at[pl.ds(b * K, K)], rows0)
      pltpu.sync_copy(rows0, h_out.at[cid, pl.ds(b * K, K)])
    return _
  lax.fori_loop(0, (NB + NS - 1) // NS, _pub, None)


_scb = pl.kernel(
    _scb_body,
    out_type=jax.ShapeDtypeStruct((NC, N, F), jnp.float32),
    mesh=_mesh,
    compiler_params=pltpu.CompilerParams(needs_layout_passes=False),
    scratch_types=[
        pltpu.VMEM((K, F), jnp.float32),
        pltpu.VMEM((K, F), jnp.float32),
        pltpu.VMEM((K,), jnp.int32),
        pltpu.VMEM((K,), jnp.int32),
        pltpu.VMEM((K,), jnp.int32),
        pltpu.VMEM((K,), jnp.int32),
        pltpu.VMEM((K,), jnp.int32),
        pltpu.VMEM((K,), jnp.int32),
        pltpu.VMEM((K,), jnp.float32),
        pltpu.VMEM((K,), jnp.float32),
        pltpu.VMEM((K,), jnp.int32),
        pltpu.VMEM((K,), jnp.int32),
        pltpu.VMEM((K,), jnp.float32),
        pltpu.VMEM((K,), jnp.float32),
        pltpu.VMEM((K,), jnp.float32),
        pltpu.VMEM((K,), jnp.float32),
        pltpu.SemaphoreType.DMA,
        pltpu.SemaphoreType.DMA,
        pltpu.SemaphoreType.DMA,
        pltpu.SemaphoreType.DMA,
        pltpu.SemaphoreType.DMA,
        pltpu.SemaphoreType.DMA,
        pltpu.VMEM_SHARED((N, F), jnp.float32),
    ],
)


def _tc2_body(nh_ref, hp_ref, wo_ref, b_ref, out_ref):
  wo = wo_ref[...]
  h = hp_ref[0] + hp_ref[1]
  acc = lax.dot_general(nh_ref[...], wo[:, :F],
                        (((1,), (1,)), ((), ())),
                        preferred_element_type=jnp.float32)
  acc = acc + lax.dot_general(h, wo[:, F:],
                              (((1,), (1,)), ((), ())),
                              preferred_element_type=jnp.float32)
  out_ref[...] = jnp.maximum(acc + b_ref[...], 0.0)


def _tc2(nh, h_parts, wo, b2d):
  blk = 1000
  return pl.pallas_call(
      _tc2_body,
      grid=(N // blk,),
      in_specs=[
          pl.BlockSpec((blk, F), lambda i: (i, 0)),
          pl.BlockSpec((NC, blk, F), lambda i: (0, i, 0)),
          pl.BlockSpec((F, 2 * F), lambda i: (0, 0)),
          pl.BlockSpec((1, F), lambda i: (0, 0)),
      ],
      out_specs=pl.BlockSpec((blk, F), lambda i: (i, 0)),
      out_shape=jax.ShapeDtypeStruct((N, F), jnp.float32),
  )(nh, h_parts, wo, b2d)


@jax.jit
def kernel(nh, W_nf, W_attn, W_out, b_out, edge_y, edge_index):
  src = edge_index[0].astype(jnp.int32)
  dst = edge_index[1].astype(jnp.int32)
  ty = edge_y.astype(jnp.int32)
  a = jnp.stack([W_attn[0, :F], W_attn[0, F:]], axis=1)
  g, pq = _tc1(nh, W_nf, a)
  w, s_parts = _sca(src, dst, ty, pq.reshape(-1))
  s_total = _scs(s_parts)
  h_parts = _scb(src, dst, ty, w, s_total, g)
  return _tc2(nh, h_parts, W_out, b_out.reshape(1, F))


# speedup vs baseline: 117.0466x; 1.0681x over previous
import functools

import jax
import jax.numpy as jnp
from jax import lax
from jax.experimental import pallas as pl
from jax.experimental.pallas import tpu as pltpu
from jax.experimental.pallas import tpu_sc as plsc

N = 10000
F = 128
E = 320000
T = 5
NSEG = 50176
NC = 2
NS = 16
NW = NC * NS
EPW = E // NW
K = 80
NCHUNK = EPW // K
SEG_SL = NSEG // NS
ROW_SL = N // NS

_mesh = plsc.VectorSubcoreMesh(
    core_axis_name="c", subcore_axis_name="s", num_cores=NC, num_subcores=NS)


def _tc1_body(nh_ref, wnf_ref, a_ref, g_ref, pq_ref):
  g = lax.dot_general(nh_ref[...], wnf_ref[...],
                      (((1,), (1,)), ((), ())),
                      preferred_element_type=jnp.float32)
  g_ref[...] = g
  pq_ref[...] = lax.dot_general(g, a_ref[...],
                                (((1,), (0,)), ((), ())),
                                preferred_element_type=jnp.float32)


def _tc1(nh, wnf, a):
  blk = 1000
  return pl.pallas_call(
      _tc1_body,
      grid=(N // blk,),
      in_specs=[
          pl.BlockSpec((blk, F), lambda i: (i, 0)),
          pl.BlockSpec((F, F), lambda i: (0, 0)),
          pl.BlockSpec((F, 2), lambda i: (0, 0)),
      ],
      out_specs=[
          pl.BlockSpec((blk, F), lambda i: (i, 0)),
          pl.BlockSpec((blk, 2), lambda i: (i, 0)),
      ],
      out_shape=[
          jax.ShapeDtypeStruct((N, F), jnp.float32),
          jax.ShapeDtypeStruct((N, 2), jnp.float32),
      ],
  )(nh, wnf, a)


KA = 400
NCH_A = EPW // KA
KSUB = 80


def _sca_body(src_hbm, dst_hbm, ty_hbm, pq_hbm,
              w_hbm, s_out,
              pq_v, src0, src1, dst0, dst1, tyv0, tyv1, wv0, wv1, zb_v,
              kb00, kb01, kb02, kb03, kb04,
              kb10, kb11, kb12, kb13, kb14,
              seml0, seml1, semw0, semw1, sems0, sems1, s_sh):
  cid = lax.axis_index("c")
  sid = lax.axis_index("s")
  wid = sid * NC + cid
  src_v = [src0, src1]
  dst_v = [dst0, dst1]
  ty_v = [tyv0, tyv1]
  w_v = [wv0, wv1]
  kbufs = [[kb00, kb01, kb02, kb03, kb04],
           [kb10, kb11, kb12, kb13, kb14]]
  sem_l = [seml0, seml1]
  sem_w = [semw0, semw1]
  sem_s = [sems0, sems1]

  def _z(i, _):
    zb_v[pl.ds(i * 16, 16)] = jnp.zeros((16,), jnp.float32)
    return _
  lax.fori_loop(0, SEG_SL // 16, _z, None)
  pltpu.sync_copy(zb_v, s_sh.at[pl.ds(sid * SEG_SL, SEG_SL)])

  pltpu.sync_copy(pq_hbm, pq_v)
  plsc.subcore_barrier()

  def _issue_loads(c, b):
    base = wid * EPW + c * KA
    pltpu.async_copy(src_hbm.at[pl.ds(base, KA)], src_v[b], sem_l[b])
    pltpu.async_copy(dst_hbm.at[pl.ds(base, KA)], dst_v[b], sem_l[b])
    pltpu.async_copy(ty_hbm.at[pl.ds(base, KA)], ty_v[b], sem_l[b])

  def _drain_loads(b):
    pltpu.make_async_copy(src_hbm.at[pl.ds(0, KA)], src_v[b], sem_l[b]).wait()
    pltpu.make_async_copy(dst_hbm.at[pl.ds(0, KA)], dst_v[b], sem_l[b]).wait()
    pltpu.make_async_copy(ty_hbm.at[pl.ds(0, KA)], ty_v[b], sem_l[b]).wait()

  def _compute(b):
    for j in range(KA // 16):
      sl = pl.ds(j * 16, 16)
      s16 = src_v[b][sl]
      d16 = dst_v[b][sl]
      t16 = ty_v[b][sl]
      pe = plsc.load_gather(pq_v, [s16 * 2])
      qe = plsc.load_gather(pq_v, [d16 * 2 + 1])
      ef = pe + qe
      ef = jnp.where(ef > 0.0, ef, 0.2 * ef)
      w_v[b][sl] = jnp.exp(ef)
      kbufs[b][j // (KSUB // 16)][pl.ds((j % (KSUB // 16)) * 16, 16)] = (
          d16 * T + t16)

  def _issue_outs(c, b):
    base = wid * EPW + c * KA
    pltpu.async_copy(w_v[b], w_hbm.at[pl.ds(base, KA)], sem_w[b])
    for i in range(KA // KSUB):
      pltpu.async_copy(w_v[b].at[pl.ds(i * KSUB, KSUB)],
                       s_sh.at[kbufs[b][i]], sem_s[b], add=True)

  def _drain_outs(b):
    pltpu.make_async_copy(w_v[b], w_hbm.at[pl.ds(0, KA)], sem_w[b]).wait()
    for i in range(KA // KSUB):
      pltpu.make_async_copy(w_v[b].at[pl.ds(i * KSUB, KSUB)],
                            s_sh.at[kbufs[b][i]], sem_s[b]).wait()

  _issue_loads(0, 0)
  _issue_loads(1, 1)

  def _main(i, _):
    for b in range(2):
      c = 2 * i + b
      @pl.when(c < NCH_A)
      def _():
        _drain_loads(b)
        @pl.when(c >= 2)
        def _():
          _drain_outs(b)
        _compute(b)
        _issue_outs(c, b)
        cn = c + 2
        @pl.when(cn < NCH_A)
        def _():
          _issue_loads(cn, b)
    return _
  lax.fori_loop(0, (NCH_A + 1) // 2, _main, None)
  _drain_outs((NCH_A - 1) % 2)
  _drain_outs(NCH_A % 2)

  plsc.subcore_barrier()
  pltpu.sync_copy(s_sh.at[pl.ds(sid * SEG_SL, SEG_SL)], zb_v)
  pltpu.sync_copy(zb_v, s_out.at[pl.ds(cid * NSEG + sid * SEG_SL, SEG_SL)])


_sca = pl.kernel(
    _sca_body,
    out_type=(
        jax.ShapeDtypeStruct((E,), jnp.float32),
        jax.ShapeDtypeStruct((NC * NSEG,), jnp.float32),
    ),
    mesh=_mesh,
    compiler_params=pltpu.CompilerParams(needs_layout_passes=False),
    scratch_types=(
        [pltpu.VMEM((N * 2,), jnp.float32)]
        + [pltpu.VMEM((KA,), jnp.int32)] * 6
        + [pltpu.VMEM((KA,), jnp.float32)] * 2
        + [pltpu.VMEM((SEG_SL,), jnp.float32)]
        + [pltpu.VMEM((KSUB,), jnp.int32)] * 10
        + [pltpu.SemaphoreType.DMA] * 6
        + [pltpu.VMEM_SHARED((NSEG,), jnp.float32)]
    ),
)


def _scb_body(src_hbm, dst_hbm, ty_hbm, w_hbm, sp_hbm, g_hbm,
              h_out, s_hbm,
              rows0, rows1, src0, src1, dst0, dst1, ty0, ty1,
              w0, w1, key0, key1, sv0, sv1, al0, al1, b0_v, b1_v,
              seml0, seml1, semg0, semg1, sems0, sems1, h_sh):
  cid = lax.axis_index("c")
  sid = lax.axis_index("s")
  wid = sid * NC + cid
  rows = [rows0, rows1]
  srcs = [src0, src1]
  dsts = [dst0, dst1]
  tys = [ty0, ty1]
  ws = [w0, w1]
  keys = [key0, key1]
  svs = [sv0, sv1]
  als = [al0, al1]
  seml = [seml0, seml1]
  semg = [semg0, semg1]
  sems = [sems0, sems1]

  def _z(i, _):
    r = i // (F // 16)
    cc = i % (F // 16)
    rows0[r, pl.ds(cc * 16, 16)] = jnp.zeros((16,), jnp.float32)
    return _
  lax.fori_loop(0, K * (F // 16), _z, None)
  NB = N // K
  def _zh(i, _):
    b = sid + i * NS
    @pl.when(b < NB)
    def _():
      pltpu.sync_copy(rows0, h_sh.at[pl.ds(b * K, K)])
    return _
  lax.fori_loop(0, (NB + NS - 1) // NS, _zh, None)

  sbase = sid * SEG_SL
  pltpu.sync_copy(sp_hbm.at[pl.ds(sbase, SEG_SL)], b0_v)
  pltpu.sync_copy(sp_hbm.at[pl.ds(NSEG + sbase, SEG_SL)], b1_v)
  def _acc(i, _):
    sl = pl.ds(i * 16, 16)
    b0_v[sl] = b0_v[sl] + b1_v[sl]
    return _
  lax.fori_loop(0, SEG_SL // 16, _acc, None)
  pltpu.sync_copy(b0_v, s_hbm.at[pl.ds(sbase, SEG_SL)])
  plsc.subcore_barrier()

  def _issue_loads(c, b):
    base = wid * EPW + c * K
    pltpu.async_copy(src_hbm.at[pl.ds(base, K)], srcs[b], seml[b])
    pltpu.async_copy(dst_hbm.at[pl.ds(base, K)], dsts[b], seml[b])
    pltpu.async_copy(ty_hbm.at[pl.ds(base, K)], tys[b], seml[b])
    pltpu.async_copy(w_hbm.at[pl.ds(base, K)], ws[b], seml[b])

  def _drain_loads(b):
    pltpu.make_async_copy(src_hbm.at[pl.ds(0, K)], srcs[b], seml[b]).wait()
    pltpu.make_async_copy(dst_hbm.at[pl.ds(0, K)], dsts[b], seml[b]).wait()
    pltpu.make_async_copy(ty_hbm.at[pl.ds(0, K)], tys[b], seml[b]).wait()
    pltpu.make_async_copy(w_hbm.at[pl.ds(0, K)], ws[b], seml[b]).wait()

  def _issue_gathers(b):
    for j in range(K // 16):
      sl = pl.ds(j * 16, 16)
      keys[b][sl] = dsts[b][sl] * T + tys[b][sl]
    pltpu.async_copy(g_hbm.at[srcs[b]], rows[b], semg[b])
    pltpu.async_copy(s_hbm.at[keys[b]], svs[b], semg[b])

  def _drain_gathers(b):
    pltpu.make_async_copy(g_hbm.at[pl.ds(0, K)], rows[b], semg[b]).wait()
    pltpu.make_async_copy(s_hbm.at[pl.ds(0, K)], svs[b], semg[b]).wait()

  def _drain_scatter(b):
    pltpu.make_async_copy(g_hbm.at[pl.ds(0, K)], rows[b], sems[b]).wait()

  def _compute_scatter(b):
    for j in range(K // 16):
      sl = pl.ds(j * 16, 16)
      als[b][sl] = ws[b][sl] / svs[b][sl]
    def _scale(j, _2):
      al16 = als[b][pl.ds(j * 16, 16)]
      for l in range(16):
        a = al16[l]
        e = j * 16 + l
        for cc in range(F // 16):
          csl = pl.ds(cc * 16, 16)
          rows[b][e, csl] = rows[b][e, csl] * a
      return _2
    lax.fori_loop(0, K // 16, _scale, None)
    pltpu.async_copy(rows[b], h_sh.at[dsts[b]], sems[b], add=True)

  for b in range(2):
    _issue_loads(b, b)
    _drain_loads(b)
    _issue_gathers(b)

  def _main(i, _):
    for b in range(2):
      c = 2 * i + b
      @pl.when(c < NCHUNK)
      def _():
        _drain_gathers(b)
        _compute_scatter(b)
        cn = c + 2
        @pl.when(cn < NCHUNK)
        def _():
          _issue_loads(cn, b)
          _drain_loads(b)
          _drain_scatter(b)
          _issue_gathers(b)
    return _
  lax.fori_loop(0, (NCHUNK + 1) // 2, _main, None)

  _drain_scatter((NCHUNK - 1) % 2)
  _drain_scatter(NCHUNK % 2)

  plsc.subcore_barrier()
  def _pub(i, _):
    b = sid + i * NS
    @pl.when(b < NB)
    def _():
      pltpu.sync_copy(h_sh.at[pl.ds(b * K, K)], rows0)
      pltpu.sync_copy(rows0, h_out.at[cid, pl.ds(b * K, K)])
    return _
  lax.fori_loop(0, (NB + NS - 1) // NS, _pub, None)


_scb = pl.kernel(
    _scb_body,
    out_type=(
        jax.ShapeDtypeStruct((NC, N, F), jnp.float32),
        jax.ShapeDtypeStruct((NSEG,), jnp.float32),
    ),
    mesh=_mesh,
    compiler_params=pltpu.CompilerParams(needs_layout_passes=False),
    scratch_types=(
        [pltpu.VMEM((K, F), jnp.float32)] * 2
        + [pltpu.VMEM((K,), jnp.int32)] * 6
        + [pltpu.VMEM((K,), jnp.float32)] * 2
        + [pltpu.VMEM((K,), jnp.int32)] * 2
        + [pltpu.VMEM((K,), jnp.float32)] * 4
        + [pltpu.VMEM((SEG_SL,), jnp.float32)] * 2
        + [pltpu.SemaphoreType.DMA] * 6
        + [pltpu.VMEM_SHARED((N, F), jnp.float32)]
    ),
)


def _tc2_body(nh_ref, hp_ref, wo_ref, b_ref, out_ref):
  wo = wo_ref[...]
  h = hp_ref[0] + hp_ref[1]
  acc = lax.dot_general(nh_ref[...], wo[:, :F],
                        (((1,), (1,)), ((), ())),
                        preferred_element_type=jnp.float32)
  acc = acc + lax.dot_general(h, wo[:, F:],
                              (((1,), (1,)), ((), ())),
                              preferred_element_type=jnp.float32)
  out_ref[...] = jnp.maximum(acc + b_ref[...], 0.0)


def _tc2(nh, h_parts, wo, b2d):
  blk = 1000
  return pl.pallas_call(
      _tc2_body,
      grid=(N // blk,),
      in_specs=[
          pl.BlockSpec((blk, F), lambda i: (i, 0)),
          pl.BlockSpec((NC, blk, F), lambda i: (0, i, 0)),
          pl.BlockSpec((F, 2 * F), lambda i: (0, 0)),
          pl.BlockSpec((1, F), lambda i: (0, 0)),
      ],
      out_specs=pl.BlockSpec((blk, F), lambda i: (i, 0)),
      out_shape=jax.ShapeDtypeStruct((N, F), jnp.float32),
  )(nh, h_parts, wo, b2d)


@jax.jit
def kernel(nh, W_nf, W_attn, W_out, b_out, edge_y, edge_index):
  src = edge_index[0].astype(jnp.int32)
  dst = edge_index[1].astype(jnp.int32)
  ty = edge_y.astype(jnp.int32)
  a = jnp.stack([W_attn[0, :F], W_attn[0, F:]], axis=1)
  g, pq = _tc1(nh, W_nf, a)
  w, s_parts = _sca(src, dst, ty, pq.reshape(-1))
  h_parts, _unused = _scb(src, dst, ty, w, s_parts, g)
  return _tc2(nh, h_parts, W_out, b_out.reshape(1, F))
